# baseline (speedup 1.0000x reference)
import functools
import math

import jax
import jax.numpy as jnp
from jax import lax
from jax.experimental import pallas as pl

_H = 10
_SUB = 0.8


def _routing_body(rnd_ref, u_ref, out_ref, *, O, H, I, D, subset):
    R = rnd_ref[0]
    m = jnp.transpose((R * jnp.float32(2.0**23)).astype(jnp.int32))
    rows = O * H
    nsub = jnp.float32(subset)

    lo = jnp.zeros((rows, 1), jnp.int32)
    hi = jnp.full((rows, 1), 2**23, jnp.int32)
    for _ in range(23):
        mid = (lo + hi) >> 1
        cnt = jnp.sum((m >= mid).astype(jnp.float32), axis=1, keepdims=True)
        pred = cnt >= nsub
        lo = jnp.where(pred, mid, lo)
        hi = jnp.where(pred, hi, mid)
    T = lo

    gt = m > T
    eq = m == T
    cnt_gt = jnp.sum(gt.astype(jnp.float32), axis=1, keepdims=True)
    need = nsub - cnt_gt
    col = lax.broadcasted_iota(jnp.int32, (rows, I), 1)
    lo2 = jnp.full((rows, 1), -1, jnp.int32)
    hi2 = jnp.full((rows, 1), I - 1, jnp.int32)
    for _ in range(11):
        mid = (lo2 + hi2) >> 1
        cnt = jnp.sum(
            jnp.where(eq & (col <= mid), 1.0, 0.0), axis=1, keepdims=True)
        ok = cnt >= need
        hi2 = jnp.where(ok, mid, hi2)
        lo2 = jnp.where(ok, lo2, mid)
    r = jnp.where(gt | (eq & (col <= hi2)), jnp.float32(1.0),
                  jnp.float32(0.0))

    u = u_ref[0]
    for o in range(O):
        uo = u[o]
        n2 = jnp.sum(uo * uo, axis=0, keepdims=True)
        n = jnp.sqrt(jnp.maximum(n2, 1e-24))
        rn = r[o * H:(o + 1) * H] * n
        S = jnp.sum(rn, axis=1, keepdims=True)
        Nmat = jnp.sum(uo[:, None, :] * rn[None, :, :], axis=2)
        Srow = jnp.concatenate([S[h:h + 1, 0:1] for h in range(H)], axis=1)
        mu_mat = Nmat / Srow
        losses = []
        for h in range(H):
            diff = uo - mu_mat[:, h:h + 1]
            d2 = jnp.sum(diff * diff, axis=0, keepdims=True)
            term = jnp.sqrt(jnp.maximum(d2, 1e-24))
            losses.append(jnp.sum(term, axis=1, keepdims=True))
        lvec = jnp.concatenate(losses, axis=1)
        lmin = jnp.min(lvec, axis=1, keepdims=True)
        hio = lax.broadcasted_iota(jnp.int32, (1, H), 1)
        pick = jnp.min(jnp.where(lvec == lmin, hio, H), axis=1,
                       keepdims=True)
        acc = jnp.zeros((D, 1), jnp.float32)
        for h in range(H):
            acc = acc + (pick == h).astype(jnp.float32) * mu_mat[:, h:h + 1]
        out_ref[0, :, o:o + 1] = acc


def _run(u_predict, interpret=False):
    B, I, O, D = u_predict.shape
    H = _H
    subset = math.ceil(_SUB * I)
    rand_key = jax.random.fold_in(jax.random.key(0), 1)
    rnd = jax.random.uniform(rand_key, (B, I, O, H), dtype=jnp.float32)
    rnd_r = rnd.reshape(B, I, O * H)
    u_t = jnp.transpose(u_predict, (0, 2, 3, 1))
    body = functools.partial(_routing_body, O=O, H=H, I=I, D=D, subset=subset)
    vt = pl.pallas_call(
        body,
        grid=(B,),
        in_specs=[
            pl.BlockSpec((1, I, O * H), lambda b: (b, 0, 0)),
            pl.BlockSpec((1, O, D, I), lambda b: (b, 0, 0, 0)),
        ],
        out_specs=pl.BlockSpec((1, D, O), lambda b: (b, 0, 0)),
        out_shape=jax.ShapeDtypeStruct((B, D, O), jnp.float32),
        interpret=interpret,
    )(rnd_r, u_t)
    return jnp.transpose(vt, (0, 2, 1))


def kernel(u_predict):
    return _run(u_predict)


# speedup vs baseline: 1.5613x; 1.5613x over previous
import functools
import math

import jax
import jax.numpy as jnp
from jax import lax
from jax.experimental import pallas as pl
from jax.experimental.pallas import tpu as pltpu

_H = 10
_SUB = 0.8


def _routing_body(rnd_ref, u_ref, out_ref, r_scratch, *, O, H, I, D, subset):
    R = rnd_ref[0]
    m = jnp.transpose((R * jnp.float32(2.0**23)).astype(jnp.int32))
    rows = O * H
    nsub = jnp.float32(subset)

    lo = jnp.zeros((rows, 1), jnp.int32)
    hi = jnp.full((rows, 1), 2**23, jnp.int32)
    for _ in range(23):
        mid = (lo + hi) >> 1
        cnt = jnp.sum((m >= mid).astype(jnp.float32), axis=1, keepdims=True)
        pred = cnt >= nsub
        lo = jnp.where(pred, mid, lo)
        hi = jnp.where(pred, hi, mid)
    T = lo

    gt = m > T
    eq = m == T
    cnt_gt = jnp.sum(gt.astype(jnp.float32), axis=1, keepdims=True)
    need = nsub - cnt_gt
    col = lax.broadcasted_iota(jnp.int32, (rows, I), 1)
    lo2 = jnp.full((rows, 1), -1, jnp.int32)
    hi2 = jnp.full((rows, 1), I - 1, jnp.int32)
    for _ in range(11):
        mid = (lo2 + hi2) >> 1
        cnt = jnp.sum(
            jnp.where(eq & (col <= mid), 1.0, 0.0), axis=1, keepdims=True)
        ok = cnt >= need
        hi2 = jnp.where(ok, mid, hi2)
        lo2 = jnp.where(ok, lo2, mid)
    r_scratch[...] = jnp.where(gt | (eq & (col <= hi2)), jnp.float32(1.0),
                               jnp.float32(0.0))
    r = r_scratch[...]

    u = u_ref[0]
    for o in range(O):
        uo = u[o]
        n2 = jnp.sum(uo * uo, axis=0, keepdims=True)
        n = jnp.sqrt(jnp.maximum(n2, 1e-24))
        rn = r[o * H:(o + 1) * H] * n
        S = jnp.sum(rn, axis=1, keepdims=True)
        mus = []
        d2s = []
        for h in range(H):
            Nh = jnp.sum(uo * rn[h:h + 1], axis=1, keepdims=True)
            mu = Nh / S[h:h + 1]
            diff = uo - mu
            d2s.append(jnp.sum(diff * diff, axis=0, keepdims=True))
            mus.append(mu)
        term = jnp.sqrt(jnp.maximum(jnp.concatenate(d2s, axis=0), 1e-24))
        loss = jnp.sum(term, axis=1, keepdims=True)
        lmin = jnp.min(loss, axis=0, keepdims=True)
        hio = lax.broadcasted_iota(jnp.int32, (H, 1), 0)
        pick = jnp.min(jnp.where(loss == lmin, hio, H), axis=0,
                       keepdims=True)
        acc = jnp.zeros((D, 1), jnp.float32)
        for h in range(H):
            acc = acc + (pick == h).astype(jnp.float32) * mus[h]
        out_ref[0, :, o:o + 1] = acc


def _run(u_predict, interpret=False):
    B, I, O, D = u_predict.shape
    H = _H
    subset = math.ceil(_SUB * I)
    rand_key = jax.random.fold_in(jax.random.key(0), 1)
    rnd = jax.random.uniform(rand_key, (B, I, O, H), dtype=jnp.float32)
    rnd_r = rnd.reshape(B, I, O * H)
    u_t = jnp.transpose(u_predict, (0, 2, 3, 1))
    body = functools.partial(_routing_body, O=O, H=H, I=I, D=D, subset=subset)
    vt = pl.pallas_call(
        body,
        grid=(B,),
        in_specs=[
            pl.BlockSpec((1, I, O * H), lambda b: (b, 0, 0)),
            pl.BlockSpec((1, O, D, I), lambda b: (b, 0, 0, 0)),
        ],
        out_specs=pl.BlockSpec((1, D, O), lambda b: (b, 0, 0)),
        out_shape=jax.ShapeDtypeStruct((B, D, O), jnp.float32),
        scratch_shapes=[pltpu.VMEM((O * H, I), jnp.float32)],
        interpret=interpret,
    )(rnd_r, u_t)
    return jnp.transpose(vt, (0, 2, 1))


def kernel(u_predict):
    return _run(u_predict)
